# baseline (speedup 1.0000x reference)
import functools

import jax
import jax.numpy as jnp
from jax import lax
from jax.experimental import pallas as pl
from jax.experimental.pallas import tpu as pltpu
from jax.experimental.pallas import tpu_sc as plsc

_L = 16
_NW = 32
_N = 4096
_PW = _N // _NW
_D = 32
_K = 512
_G = _K // _L


def _lane_gather(vec, idx):
    return lax.gather(
        vec, idx[:, None],
        lax.GatherDimensionNumbers(
            offset_dims=(), collapsed_slice_dims=(0,), start_index_map=(0,)),
        (1,), mode=lax.GatherScatterMode.PROMISE_IN_BOUNDS)


def _splat_idx(j):
    return jnp.full((_L,), j, dtype=jnp.int32)


def _sc_body(z_hbm, e_hbm, o_hbm, z_v, e_v, o_v):
    wid = lax.axis_index("s") * 2 + lax.axis_index("c")
    pltpu.sync_copy(z_hbm.at[wid], z_v)
    pltpu.sync_copy(e_hbm, e_v)
    iota = lax.iota(jnp.int32, _L)

    def point_body(p, res):
        zrow0 = z_v[p, pl.ds(0, _L)]
        zrow1 = z_v[p, pl.ds(_L, _L)]
        zs = [_lane_gather(zrow0, _splat_idx(j)) for j in range(_L)]
        zs += [_lane_gather(zrow1, _splat_idx(j)) for j in range(_L)]
        runmin = jnp.full((_L,), jnp.inf, dtype=jnp.float32)
        runidx = jnp.zeros((_L,), dtype=jnp.int32)
        for g in range(_G):
            acc = None
            for d in range(_D):
                ev = e_v[g, d, :]
                t = zs[d] - ev
                sq = t * t
                acc = sq if acc is None else acc + sq
            lt = acc < runmin
            runmin = jnp.where(lt, acc, runmin)
            runidx = jnp.where(lt, iota + (g * _L), runidx)
        m, im = runmin, runidx
        for sh in (8, 4, 2, 1):
            rot = (iota + sh) % _L
            rm = _lane_gather(m, rot)
            ri = _lane_gather(im, rot)
            take = (rm < m) | ((rm == m) & (ri < im))
            m = jnp.where(take, rm, m)
            im = jnp.where(take, ri, im)
        lane = lax.rem(p, _L)
        res = jnp.where(iota == lane, im, res)

        @pl.when(lane == _L - 1)
        def _flush():
            o_v[pl.ds(pl.multiple_of(p - (_L - 1), _L), _L)] = res

        return res

    lax.fori_loop(0, _PW, point_body, jnp.zeros((_L,), dtype=jnp.int32))
    pltpu.sync_copy(o_v, o_hbm.at[pl.ds(wid * _PW, _PW)])


def kernel(z_e_x, emb):
    b, d, h, w = z_e_x.shape
    hw = h * w
    zw = (z_e_x.reshape(b, d, hw).transpose(0, 2, 1)
          .reshape(_NW, _PW, d))
    e3 = emb.reshape(_G, _L, d).transpose(0, 2, 1)
    mesh = plsc.VectorSubcoreMesh(core_axis_name="c", subcore_axis_name="s")
    fn = functools.partial(
        pl.kernel,
        mesh=mesh,
        compiler_params=pltpu.CompilerParams(use_tc_tiling_on_sc=False),
        out_type=jax.ShapeDtypeStruct((_N,), jnp.int32),
        scratch_types=[
            pltpu.VMEM((_PW, d), jnp.float32),
            pltpu.VMEM((_G, d, _L), jnp.float32),
            pltpu.VMEM((_PW,), jnp.int32),
        ],
    )(_sc_body)
    lat = fn(zw, e3)
    return lat.reshape(b, h, w)


# speedup vs baseline: 2.0335x; 2.0335x over previous
import functools

import jax
import jax.numpy as jnp
from jax import lax
from jax.experimental import pallas as pl
from jax.experimental.pallas import tpu as pltpu
from jax.experimental.pallas import tpu_sc as plsc

_L = 16
_NW = 32
_D = 32
_K = 512
_G = _K // _L
_B_SC = 2


def _tc_body(zt_ref, et_ref, o_ref):
    hw, d_dim = zt_ref.shape[1], zt_ref.shape[2]
    k = et_ref.shape[1]
    zt = zt_ref[0]
    et = et_ref[...]
    acc = None
    for d in range(d_dim):
        zd = zt[:, d][:, None]
        ed = et[d, :][None, :]
        diff = zd - ed
        sq = diff * diff
        acc = sq if acc is None else acc + sq
    min_val = jnp.min(acc, axis=1, keepdims=True)
    idx = jax.lax.broadcasted_iota(jnp.int32, (hw, k), 1)
    masked = jnp.where(acc == min_val, idx, k)
    o_ref[0, 0, :] = jnp.min(masked, axis=1)


def _tc_part(z_tc, emb):
    b, d, h, w = z_tc.shape
    k = emb.shape[0]
    hw = h * w
    zt = z_tc.reshape(b, d, hw).transpose(0, 2, 1)
    et = emb.T
    out = pl.pallas_call(
        _tc_body,
        grid=(b,),
        in_specs=[
            pl.BlockSpec((1, hw, d), lambda i: (i, 0, 0)),
            pl.BlockSpec((d, k), lambda i: (0, 0)),
        ],
        out_specs=pl.BlockSpec((1, 1, hw), lambda i: (i, 0, 0)),
        out_shape=jax.ShapeDtypeStruct((b, 1, hw), jnp.int32),
    )(zt, et)
    return out.reshape(b, h, w)


def _lane_gather(vec, idx):
    return lax.gather(
        vec, idx[:, None],
        lax.GatherDimensionNumbers(
            offset_dims=(), collapsed_slice_dims=(0,), start_index_map=(0,)),
        (1,), mode=lax.GatherScatterMode.PROMISE_IN_BOUNDS)


def _splat_idx(j):
    return jnp.full((_L,), j, dtype=jnp.int32)


def _sc_body(pw, z_hbm, e_hbm, o_hbm, z_v, e_v, o_v):
    wid = lax.axis_index("s") * 2 + lax.axis_index("c")
    pltpu.sync_copy(z_hbm.at[wid], z_v)
    pltpu.sync_copy(e_hbm, e_v)
    iota = lax.iota(jnp.int32, _L)

    def point_body(p, res):
        zrow0 = z_v[p, pl.ds(0, _L)]
        zrow1 = z_v[p, pl.ds(_L, _L)]
        zs = [_lane_gather(zrow0, _splat_idx(j)) for j in range(_L)]
        zs += [_lane_gather(zrow1, _splat_idx(j)) for j in range(_L)]
        runmin = jnp.full((_L,), jnp.inf, dtype=jnp.float32)
        runidx = jnp.zeros((_L,), dtype=jnp.int32)
        for g in range(_G):
            acc = None
            for d in range(_D):
                ev = e_v[g, d, :]
                t = zs[d] - ev
                sq = t * t
                acc = sq if acc is None else acc + sq
            lt = acc < runmin
            runmin = jnp.where(lt, acc, runmin)
            runidx = jnp.where(lt, iota + (g * _L), runidx)
        m, im = runmin, runidx
        for sh in (8, 4, 2, 1):
            rot = (iota + sh) % _L
            rm = _lane_gather(m, rot)
            ri = _lane_gather(im, rot)
            take = (rm < m) | ((rm == m) & (ri < im))
            m = jnp.where(take, rm, m)
            im = jnp.where(take, ri, im)
        lane = lax.rem(p, _L)
        res = jnp.where(iota == lane, im, res)

        @pl.when(lane == _L - 1)
        def _flush():
            o_v[pl.ds(pl.multiple_of(p - (_L - 1), _L), _L)] = res

        return res

    lax.fori_loop(0, pw, point_body, jnp.zeros((_L,), dtype=jnp.int32))
    pltpu.sync_copy(o_v, o_hbm.at[pl.ds(wid * pw, pw)])


def _sc_part(z_sc, emb):
    b, d, h, w = z_sc.shape
    hw = h * w
    n = b * hw
    pw = n // _NW
    zw = (z_sc.reshape(b, d, hw).transpose(0, 2, 1).reshape(_NW, pw, d))
    e3 = emb.reshape(_G, _L, d).transpose(0, 2, 1)
    mesh = plsc.VectorSubcoreMesh(core_axis_name="c", subcore_axis_name="s")
    fn = functools.partial(
        pl.kernel,
        mesh=mesh,
        compiler_params=pltpu.CompilerParams(use_tc_tiling_on_sc=False),
        out_type=jax.ShapeDtypeStruct((n,), jnp.int32),
        scratch_types=[
            pltpu.VMEM((pw, d), jnp.float32),
            pltpu.VMEM((_G, d, _L), jnp.float32),
            pltpu.VMEM((pw,), jnp.int32),
        ],
    )(functools.partial(_sc_body, pw))
    return fn(zw, e3).reshape(b, h, w)


def kernel(z_e_x, emb):
    b = z_e_x.shape[0]
    b_tc = b - _B_SC
    lat_sc = _sc_part(z_e_x[b_tc:], emb)
    lat_tc = _tc_part(z_e_x[:b_tc], emb)
    return jnp.concatenate([lat_tc, lat_sc], axis=0)


# speedup vs baseline: 2.0357x; 1.0011x over previous
import functools

import jax
import jax.numpy as jnp
from jax import lax
from jax.experimental import pallas as pl
from jax.experimental.pallas import tpu as pltpu
from jax.experimental.pallas import tpu_sc as plsc

_L = 16
_NW = 32
_D = 32
_K = 512
_G = _K // _L
_B_SC = 2


def _tc_body(zt_ref, et_ref, o_ref):
    hw, d_dim = zt_ref.shape[1], zt_ref.shape[2]
    k = et_ref.shape[1]
    zt = zt_ref[0]
    et = et_ref[...]
    acc = None
    for d in range(d_dim):
        zd = zt[:, d][:, None]
        ed = et[d, :][None, :]
        diff = zd - ed
        sq = diff * diff
        acc = sq if acc is None else acc + sq
    min_val = jnp.min(acc, axis=1, keepdims=True)
    idx = jax.lax.broadcasted_iota(jnp.int32, (hw, k), 1)
    masked = jnp.where(acc == min_val, idx, k)
    o_ref[0, 0, :] = jnp.min(masked, axis=1)


def _tc_part(z_tc, emb):
    b, d, h, w = z_tc.shape
    k = emb.shape[0]
    hw = h * w
    zt = z_tc.reshape(b, d, hw).transpose(0, 2, 1)
    et = emb.T
    out = pl.pallas_call(
        _tc_body,
        grid=(b,),
        in_specs=[
            pl.BlockSpec((1, hw, d), lambda i: (i, 0, 0)),
            pl.BlockSpec((d, k), lambda i: (0, 0)),
        ],
        out_specs=pl.BlockSpec((1, 1, hw), lambda i: (i, 0, 0)),
        out_shape=jax.ShapeDtypeStruct((b, 1, hw), jnp.int32),
    )(zt, et)
    return out.reshape(b, h, w)


def _lane_gather(vec, idx):
    return lax.gather(
        vec, idx[:, None],
        lax.GatherDimensionNumbers(
            offset_dims=(), collapsed_slice_dims=(0,), start_index_map=(0,)),
        (1,), mode=lax.GatherScatterMode.PROMISE_IN_BOUNDS)


def _splat_idx(j):
    return jnp.full((_L,), j, dtype=jnp.int32)


def _sc_body(pw, z_hbm, e_hbm, o_hbm, z_v, e_v, o_v):
    wid = lax.axis_index("s") * 2 + lax.axis_index("c")
    pltpu.sync_copy(z_hbm.at[wid], z_v)
    pltpu.sync_copy(e_hbm, e_v)
    iota = lax.iota(jnp.int32, _L)

    def point_body(p, res):
        zrow0 = z_v[p, pl.ds(0, _L)]
        zrow1 = z_v[p, pl.ds(_L, _L)]
        zs = [_lane_gather(zrow0, _splat_idx(j)) for j in range(_L)]
        zs += [_lane_gather(zrow1, _splat_idx(j)) for j in range(_L)]
        runmin = jnp.full((_L,), jnp.inf, dtype=jnp.float32)
        runidx = jnp.zeros((_L,), dtype=jnp.int32)
        for g in range(_G):
            acc = None
            for d in range(_D):
                ev = e_v[g, d, :]
                t = zs[d] - ev
                sq = t * t
                acc = sq if acc is None else acc + sq
            lt = acc < runmin
            runmin = jnp.where(lt, acc, runmin)
            runidx = jnp.where(lt, iota + (g * _L), runidx)
        m, im = runmin, runidx
        for sh in (8, 4, 2, 1):
            rot = (iota + sh) % _L
            rm = _lane_gather(m, rot)
            ri = _lane_gather(im, rot)
            take = (rm < m) | ((rm == m) & (ri < im))
            m = jnp.where(take, rm, m)
            im = jnp.where(take, ri, im)
        lane = lax.rem(p, _L)
        res = jnp.where(iota == lane, im, res)

        @pl.when(lane == _L - 1)
        def _flush():
            o_v[pl.ds(pl.multiple_of(p - (_L - 1), _L), _L)] = res

        return res

    lax.fori_loop(0, pw, point_body, jnp.zeros((_L,), dtype=jnp.int32))
    pltpu.sync_copy(o_v, o_hbm.at[pl.ds(wid * pw, pw)])


def _sc_part(z_sc, emb):
    b, d, h, w = z_sc.shape
    hw = h * w
    n = b * hw
    pw = n // _NW
    zw = (z_sc.reshape(b, d, hw).transpose(0, 2, 1).reshape(_NW, pw, d))
    e3 = emb.reshape(_G, _L, d).transpose(0, 2, 1)
    mesh = plsc.VectorSubcoreMesh(core_axis_name="c", subcore_axis_name="s")
    fn = functools.partial(
        pl.kernel,
        mesh=mesh,
        compiler_params=pltpu.CompilerParams(
            use_tc_tiling_on_sc=False, skip_device_barrier=True),
        out_type=jax.ShapeDtypeStruct((n,), jnp.int32),
        scratch_types=[
            pltpu.VMEM((pw, d), jnp.float32),
            pltpu.VMEM((_G, d, _L), jnp.float32),
            pltpu.VMEM((pw,), jnp.int32),
        ],
    )(functools.partial(_sc_body, pw))
    return fn(zw, e3).reshape(b, h, w)


def kernel(z_e_x, emb):
    b = z_e_x.shape[0]
    b_tc = b - _B_SC
    lat_sc = _sc_part(z_e_x[b_tc:], emb)
    lat_tc = _tc_part(z_e_x[:b_tc], emb)
    return jnp.concatenate([lat_tc, lat_sc], axis=0)


# speedup vs baseline: 2.5863x; 1.2705x over previous
import jax
import jax.numpy as jnp
from jax.experimental import pallas as pl
from jax.experimental.pallas import tpu as pltpu


def _vq_body(z_ref, et_ref, o_ref, zt_s):
    b, d_dim, hw = z_ref.shape
    k = et_ref.shape[1]
    pid = pl.program_id(0)

    @pl.when(pid == 0)
    def _relayout():
        for i in range(b):
            zt_s[i] = z_ref[i].T

    @pl.when(pid > 0)
    def _compute():
        i = pid - 1
        zt = zt_s[i]
        et = et_ref[...]
        acc = None
        for d in range(d_dim):
            zd = zt[:, d][:, None]
            ed = et[d, :][None, :]
            diff = zd - ed
            sq = diff * diff
            acc = sq if acc is None else acc + sq
        min_val = jnp.min(acc, axis=1, keepdims=True)
        idx = jax.lax.broadcasted_iota(jnp.int32, (hw, k), 1)
        masked = jnp.where(acc == min_val, idx, k)
        o_ref[0, 0, :] = jnp.min(masked, axis=1)


def kernel(z_e_x, emb):
    b, d, h, w = z_e_x.shape
    k = emb.shape[0]
    hw = h * w
    z3 = z_e_x.reshape(b, d, hw)
    et = emb.T
    out = pl.pallas_call(
        _vq_body,
        grid=(b + 1,),
        in_specs=[
            pl.BlockSpec((b, d, hw), lambda i: (0, 0, 0)),
            pl.BlockSpec((d, k), lambda i: (0, 0)),
        ],
        out_specs=pl.BlockSpec(
            (1, 1, hw), lambda i: (jnp.maximum(i - 1, 0), 0, 0)),
        out_shape=jax.ShapeDtypeStruct((b, 1, hw), jnp.int32),
        scratch_shapes=[pltpu.VMEM((b, hw, d), jnp.float32)],
    )(z3, et)
    return out.reshape(b, h, w)


# speedup vs baseline: 2.8287x; 1.0937x over previous
import jax
import jax.numpy as jnp
from jax.experimental import pallas as pl


def _vq_body(zt_ref, et_ref, o_ref):
    hw, d_dim = zt_ref.shape[1], zt_ref.shape[2]
    k = et_ref.shape[1]
    zt = zt_ref[0]
    et = et_ref[...]
    acc = None
    for d in range(d_dim):
        zd = zt[:, d][:, None]
        ed = et[d, :][None, :]
        diff = zd - ed
        sq = diff * diff
        acc = sq if acc is None else acc + sq
    min_val = jnp.min(acc, axis=1, keepdims=True)
    idx = jax.lax.broadcasted_iota(jnp.int32, (hw, k), 1)
    masked = jnp.where(acc == min_val, idx, k)
    o_ref[0, 0, :] = jnp.min(masked, axis=1)


def kernel(z_e_x, emb):
    b, d, h, w = z_e_x.shape
    k = emb.shape[0]
    hw = h * w
    zt = z_e_x.reshape(b, d, hw).transpose(0, 2, 1)
    et = emb.T
    out = pl.pallas_call(
        _vq_body,
        grid=(b,),
        in_specs=[
            pl.BlockSpec((1, hw, d), lambda i: (i, 0, 0)),
            pl.BlockSpec((d, k), lambda i: (0, 0)),
        ],
        out_specs=pl.BlockSpec((1, 1, hw), lambda i: (i, 0, 0)),
        out_shape=jax.ShapeDtypeStruct((b, 1, hw), jnp.int32),
    )(zt, et)
    return out.reshape(b, h, w)


# speedup vs baseline: 3.1310x; 1.1069x over previous
import jax
import jax.numpy as jnp
from jax.experimental import pallas as pl


def _vq_body(zt_ref, et_ref, o_ref):
    hw, d_dim = zt_ref.shape[1], zt_ref.shape[2]
    k = et_ref.shape[1]
    zt = zt_ref[0]
    et = et_ref[...]
    acc = None
    for d in range(d_dim):
        zd = zt[:, d][:, None]
        ed = et[d, :][None, :]
        diff = zd - ed
        sq = diff * diff
        acc = sq if acc is None else acc + sq
    min_val = jnp.min(acc, axis=1, keepdims=True)
    idx = jax.lax.broadcasted_iota(jnp.int32, (hw, k), 1)
    masked = jnp.where(acc == min_val, idx, k)
    res = jnp.min(masked, axis=1)
    o_ref[0] = res.reshape(o_ref.shape[1], o_ref.shape[2])


def kernel(z_e_x, emb):
    b, d, h, w = z_e_x.shape
    k = emb.shape[0]
    hw = h * w
    zt = z_e_x.reshape(b, d, hw).transpose(0, 2, 1)
    et = emb.T
    out = pl.pallas_call(
        _vq_body,
        grid=(b,),
        in_specs=[
            pl.BlockSpec((1, hw, d), lambda i: (i, 0, 0)),
            pl.BlockSpec((d, k), lambda i: (0, 0)),
        ],
        out_specs=pl.BlockSpec((1, h, w), lambda i: (i, 0, 0)),
        out_shape=jax.ShapeDtypeStruct((b, h, w), jnp.int32),
    )(zt, et)
    return out
